# baseline (speedup 1.0000x reference)
import functools

import jax
import jax.numpy as jnp
from jax import lax
from jax.experimental import pallas as pl
from jax.experimental.pallas import tpu as pltpu
from jax.experimental.pallas import tpu_sc as plsc

N_TOK = 16384
DM = 2048
NE = 16
TN = 2048

NW = 32
TPW = N_TOK // NW
GROUPS = TPW // 16
L = 16


def _logits_body(x_ref, w_ref, b_ref, lg_ref):
    lg_ref[...] = jax.lax.dot_general(
        x_ref[...], w_ref[...], (((1,), (1,)), ((), ())),
        preferred_element_type=jnp.float32,
    ) + b_ref[...]


def _logits_tc(x, W, b):
    grid = N_TOK // TN
    return pl.pallas_call(
        _logits_body,
        grid=(grid,),
        in_specs=[
            pl.BlockSpec((TN, DM), lambda i: (i, 0)),
            pl.BlockSpec((NE, DM), lambda i: (0, 0)),
            pl.BlockSpec((1, NE), lambda i: (0, 0)),
        ],
        out_specs=pl.BlockSpec((TN, NE), lambda i: (i, 0)),
        out_shape=jax.ShapeDtypeStruct((N_TOK, NE), jnp.float32),
        compiler_params=pltpu.CompilerParams(
            dimension_semantics=("parallel",)
        ),
    )(x, W, b.reshape(1, NE))


_mesh = plsc.VectorSubcoreMesh(core_axis_name="c", subcore_axis_name="s")


@functools.partial(
    pl.kernel,
    out_type=[
        jax.ShapeDtypeStruct((N_TOK * NE,), jnp.float32),
        jax.ShapeDtypeStruct((2 * N_TOK,), jnp.int32),
    ],
    mesh=_mesh,
    scratch_types=[
        pltpu.VMEM((TPW * NE,), jnp.float32),
        pltpu.VMEM((TPW * NE,), jnp.float32),
        pltpu.VMEM((2 * TPW,), jnp.int32),
    ],
    compiler_params=pltpu.CompilerParams(needs_layout_passes=False),
)
def _route_sc(lg_hbm, cw_hbm, idx_hbm, lg_v, cw_v, idx_v):
    wid = lax.axis_index("s") * 2 + lax.axis_index("c")
    base = wid * TPW
    pltpu.sync_copy(lg_hbm.at[pl.ds(base * NE, TPW * NE)], lg_v)

    lane = lax.iota(jnp.int32, L)
    neg = jnp.full((L,), -3.4e38, jnp.float32)
    zero_i = jnp.zeros((L,), jnp.int32)
    zero_f = jnp.zeros((L,), jnp.float32)

    def group(g, carry):
        rows = g * L + lane
        flat0 = rows * NE
        m1, m2 = neg, neg
        i1, i2 = zero_i, zero_i
        for e in range(NE):
            esp = jnp.full((L,), e, jnp.int32)
            le = plsc.load_gather(lg_v, [flat0 + e])
            gt1 = le > m1
            gt2 = le > m2
            i2 = jnp.where(gt1, i1, jnp.where(gt2, esp, i2))
            m2 = jnp.where(gt1, m1, jnp.where(gt2, le, m2))
            i1 = jnp.where(gt1, esp, i1)
            m1 = jnp.where(gt1, le, m1)
        w1 = 1.0 / (1.0 + jnp.exp(m2 - m1))
        w2 = 1.0 - w1
        for e in range(NE):
            esp = jnp.full((L,), e, jnp.int32)
            val = jnp.where(i1 == esp, w1, jnp.where(i2 == esp, w2, zero_f))
            plsc.store_scatter(cw_v, [flat0 + e], val)
        plsc.store_scatter(idx_v, [2 * rows], i1)
        plsc.store_scatter(idx_v, [2 * rows + 1], i2)
        return carry

    lax.fori_loop(0, GROUPS, group, 0)
    pltpu.sync_copy(cw_v, cw_hbm.at[pl.ds(base * NE, TPW * NE)])
    pltpu.sync_copy(idx_v, idx_hbm.at[pl.ds(2 * base, 2 * TPW)])


def kernel(x, W, b):
    logits = _logits_tc(x, W, b)
    cw_flat, idx_flat = _route_sc(logits.reshape(-1))
    return (
        cw_flat.reshape(N_TOK, NE, 1),
        idx_flat.reshape(N_TOK, 2),
        jnp.float32(0.0),
    )


# speedup vs baseline: 1.1138x; 1.1138x over previous
import functools

import jax
import jax.numpy as jnp
from jax import lax
from jax.experimental import pallas as pl
from jax.experimental.pallas import tpu as pltpu
from jax.experimental.pallas import tpu_sc as plsc

N_TOK = 16384
DM = 2048
NE = 16
TN = 2048

NW = 32
TPW = N_TOK // NW
GROUPS = TPW // 16
L = 16


def _logits_body(x_ref, w_ref, b_ref, lg_ref):
    lg = jax.lax.dot_general(
        x_ref[...], w_ref[...], (((1,), (1,)), ((), ())),
        preferred_element_type=jnp.float32,
    ) + b_ref[...]
    lg_ref[...] = lg.T


def _logits_tc(x, W, b):
    grid = N_TOK // TN
    return pl.pallas_call(
        _logits_body,
        grid=(grid,),
        in_specs=[
            pl.BlockSpec((TN, DM), lambda i: (i, 0)),
            pl.BlockSpec((NE, DM), lambda i: (0, 0)),
            pl.BlockSpec((1, NE), lambda i: (0, 0)),
        ],
        out_specs=pl.BlockSpec((NE, TN), lambda i: (0, i)),
        out_shape=jax.ShapeDtypeStruct((NE, N_TOK), jnp.float32),
        compiler_params=pltpu.CompilerParams(
            dimension_semantics=("arbitrary",)
        ),
    )(x, W, b.reshape(1, NE))


_mesh = plsc.VectorSubcoreMesh(core_axis_name="c", subcore_axis_name="s")


@functools.partial(
    pl.kernel,
    out_type=[
        jax.ShapeDtypeStruct((N_TOK, NE), jnp.float32),
        jax.ShapeDtypeStruct((2 * N_TOK,), jnp.int32),
    ],
    mesh=_mesh,
    scratch_types=[
        pltpu.VMEM((NE, TPW), jnp.float32),
        pltpu.VMEM((TPW, NE), jnp.float32),
        pltpu.VMEM((2 * TPW,), jnp.int32),
    ],
    compiler_params=pltpu.CompilerParams(needs_layout_passes=False),
)
def _route_sc(lg_hbm, cw_hbm, idx_hbm, lg_v, cw_v, idx_v):
    wid = lax.axis_index("s") * 2 + lax.axis_index("c")
    base = wid * TPW
    pltpu.sync_copy(lg_hbm.at[:, pl.ds(base, TPW)], lg_v)

    lane = lax.iota(jnp.int32, L)
    neg = jnp.full((L,), -3.4e38, jnp.float32)
    zero_i = jnp.zeros((L,), jnp.int32)
    zero_f = jnp.zeros((L,), jnp.float32)

    def group(g, carry):
        t0 = g * L
        rows = t0 + lane
        m1, m2 = neg, neg
        i1, i2 = zero_i, zero_i
        for e in range(NE):
            esp = jnp.full((L,), e, jnp.int32)
            le = lg_v[e, pl.ds(t0, L)]
            gt1 = le > m1
            gt2 = le > m2
            i2 = jnp.where(gt1, i1, jnp.where(gt2, esp, i2))
            m2 = jnp.where(gt1, m1, jnp.where(gt2, le, m2))
            i1 = jnp.where(gt1, esp, i1)
            m1 = jnp.where(gt1, le, m1)
        w1 = 1.0 / (1.0 + jnp.exp(m2 - m1))
        w2 = 1.0 - w1
        for e in range(NE):
            esp = jnp.full((L,), e, jnp.int32)
            val = jnp.where(i1 == esp, w1, jnp.where(i2 == esp, w2, zero_f))
            plsc.store_scatter(cw_v, [rows, esp], val)
        plsc.store_scatter(idx_v, [2 * rows], i1)
        plsc.store_scatter(idx_v, [2 * rows + 1], i2)
        return carry

    lax.fori_loop(0, GROUPS, group, 0)
    pltpu.sync_copy(cw_v, cw_hbm.at[pl.ds(base, TPW), :])
    pltpu.sync_copy(idx_v, idx_hbm.at[pl.ds(2 * base, 2 * TPW)])


def kernel(x, W, b):
    logits_t = _logits_tc(x, W, b)
    cw, idx_flat = _route_sc(logits_t)
    return (
        cw[..., None],
        idx_flat.reshape(N_TOK, 2),
        jnp.float32(0.0),
    )


# speedup vs baseline: 1.4174x; 1.2725x over previous
import functools

import jax
import jax.numpy as jnp
from jax import lax
from jax.experimental import pallas as pl
from jax.experimental.pallas import tpu as pltpu
from jax.experimental.pallas import tpu_sc as plsc

N_TOK = 16384
DM = 2048
NE = 16
TN = 2048

NW = 32
TPW = N_TOK // NW
GROUPS = TPW // 16
L = 16


def _logits_body(x_ref, w_ref, b_ref, lg_ref):
    lg = jax.lax.dot_general(
        x_ref[...], w_ref[...], (((1,), (1,)), ((), ())),
        preferred_element_type=jnp.float32,
    ) + b_ref[...]
    lg_ref[...] = lg.T


def _logits_tc(x, W, b):
    grid = N_TOK // TN
    return pl.pallas_call(
        _logits_body,
        grid=(grid,),
        in_specs=[
            pl.BlockSpec((TN, DM), lambda i: (i, 0)),
            pl.BlockSpec((NE, DM), lambda i: (0, 0)),
            pl.BlockSpec((1, NE), lambda i: (0, 0)),
        ],
        out_specs=pl.BlockSpec((NE, TN), lambda i: (0, i)),
        out_shape=jax.ShapeDtypeStruct((NE, N_TOK), jnp.float32),
        compiler_params=pltpu.CompilerParams(
            dimension_semantics=("arbitrary",)
        ),
    )(x, W, b.reshape(1, NE))


_mesh = plsc.VectorSubcoreMesh(core_axis_name="c", subcore_axis_name="s")


@functools.partial(
    pl.kernel,
    out_type=[
        jax.ShapeDtypeStruct((NE * N_TOK,), jnp.float32),
        jax.ShapeDtypeStruct((2 * N_TOK,), jnp.int32),
    ],
    mesh=_mesh,
    scratch_types=[
        pltpu.VMEM((NE, TPW), jnp.float32),
        pltpu.VMEM((NE, TPW), jnp.float32),
        pltpu.VMEM((TPW,), jnp.int32),
        pltpu.VMEM((TPW,), jnp.int32),
    ],
    compiler_params=pltpu.CompilerParams(needs_layout_passes=False),
)
def _route_sc(lg_hbm, cw_hbm, idx_hbm, lg_v, cw_v, i1_v, i2_v):
    wid = lax.axis_index("s") * 2 + lax.axis_index("c")
    base = wid * TPW
    pltpu.sync_copy(lg_hbm.at[:, pl.ds(base, TPW)], lg_v)

    neg = jnp.full((L,), -3.4e38, jnp.float32)
    zero_i = jnp.zeros((L,), jnp.int32)
    zero_f = jnp.zeros((L,), jnp.float32)

    def group(g, carry):
        t0 = g * L
        m1, m2 = neg, neg
        i1, i2 = zero_i, zero_i
        for e in range(NE):
            esp = jnp.full((L,), e, jnp.int32)
            le = lg_v[e, pl.ds(t0, L)]
            gt1 = le > m1
            gt2 = le > m2
            i2 = jnp.where(gt1, i1, jnp.where(gt2, esp, i2))
            m2 = jnp.where(gt1, m1, jnp.where(gt2, le, m2))
            i1 = jnp.where(gt1, esp, i1)
            m1 = jnp.where(gt1, le, m1)
        w1 = 1.0 / (1.0 + jnp.exp(m2 - m1))
        w2 = 1.0 - w1
        for e in range(NE):
            esp = jnp.full((L,), e, jnp.int32)
            val = jnp.where(i1 == esp, w1, jnp.where(i2 == esp, w2, zero_f))
            cw_v[e, pl.ds(t0, L)] = val
        i1_v[pl.ds(t0, L)] = i1
        i2_v[pl.ds(t0, L)] = i2
        return carry

    lax.fori_loop(0, GROUPS, group, 0)
    for e in range(NE):
        pltpu.sync_copy(cw_v.at[e], cw_hbm.at[pl.ds(e * N_TOK + base, TPW)])
    pltpu.sync_copy(i1_v, idx_hbm.at[pl.ds(base, TPW)])
    pltpu.sync_copy(i2_v, idx_hbm.at[pl.ds(N_TOK + base, TPW)])


def kernel(x, W, b):
    logits_t = _logits_tc(x, W, b)
    cw_flat, idx_flat = _route_sc(logits_t)
    cw = cw_flat.reshape(NE, N_TOK).T[..., None]
    idx = idx_flat.reshape(2, N_TOK).T
    return (cw, idx, jnp.float32(0.0))


# speedup vs baseline: 1.4540x; 1.0258x over previous
import functools

import jax
import jax.numpy as jnp
from jax import lax
from jax.experimental import pallas as pl
from jax.experimental.pallas import tpu as pltpu
from jax.experimental.pallas import tpu_sc as plsc

N_TOK = 16384
DM = 2048
NE = 16
TN = 2048

NW = 32
TPW = N_TOK // NW
GROUPS = TPW // 16
L = 16


def _logits_body(x_ref, w_ref, b_ref, lg_ref):
    lg = jax.lax.dot_general(
        x_ref[...], w_ref[...], (((1,), (1,)), ((), ())),
        preferred_element_type=jnp.float32,
    ) + b_ref[...]
    lg_ref[...] = lg.T


def _logits_tc(x, W, b):
    grid = N_TOK // TN
    return pl.pallas_call(
        _logits_body,
        grid=(grid,),
        in_specs=[
            pl.BlockSpec((TN, DM), lambda i: (i, 0)),
            pl.BlockSpec((NE, DM), lambda i: (0, 0)),
            pl.BlockSpec((1, NE), lambda i: (0, 0)),
        ],
        out_specs=pl.BlockSpec((NE, TN), lambda i: (0, i)),
        out_shape=jax.ShapeDtypeStruct((NE, N_TOK), jnp.float32),
        compiler_params=pltpu.CompilerParams(
            dimension_semantics=("arbitrary",)
        ),
    )(x, W, b.reshape(1, NE))


_mesh = plsc.VectorSubcoreMesh(core_axis_name="c", subcore_axis_name="s")


@functools.partial(
    pl.kernel,
    out_type=[
        jax.ShapeDtypeStruct((NE, N_TOK), jnp.float32),
        jax.ShapeDtypeStruct((2, N_TOK), jnp.int32),
    ],
    mesh=_mesh,
    scratch_types=[
        pltpu.VMEM((NE, TPW), jnp.float32),
        pltpu.VMEM((NE, TPW), jnp.float32),
        pltpu.VMEM((TPW,), jnp.int32),
        pltpu.VMEM((TPW,), jnp.int32),
    ],
    compiler_params=pltpu.CompilerParams(needs_layout_passes=False),
)
def _route_sc(lg_hbm, cw_hbm, idx_hbm, lg_v, cw_v, i1_v, i2_v):
    wid = lax.axis_index("s") * 2 + lax.axis_index("c")
    base = wid * TPW
    pltpu.sync_copy(lg_hbm.at[:, pl.ds(base, TPW)], lg_v)

    neg = jnp.full((L,), -3.4e38, jnp.float32)
    zero_i = jnp.zeros((L,), jnp.int32)
    zero_f = jnp.zeros((L,), jnp.float32)

    def group(g, carry):
        t0 = g * L
        m1, m2 = neg, neg
        i1, i2 = zero_i, zero_i
        for e in range(NE):
            esp = jnp.full((L,), e, jnp.int32)
            le = lg_v[e, pl.ds(t0, L)]
            gt1 = le > m1
            gt2 = le > m2
            i2 = jnp.where(gt1, i1, jnp.where(gt2, esp, i2))
            m2 = jnp.where(gt1, m1, jnp.where(gt2, le, m2))
            i1 = jnp.where(gt1, esp, i1)
            m1 = jnp.where(gt1, le, m1)
        w1 = 1.0 / (1.0 + jnp.exp(m2 - m1))
        w2 = 1.0 - w1
        for e in range(NE):
            esp = jnp.full((L,), e, jnp.int32)
            val = jnp.where(i1 == esp, w1, jnp.where(i2 == esp, w2, zero_f))
            cw_v[e, pl.ds(t0, L)] = val
        i1_v[pl.ds(t0, L)] = i1
        i2_v[pl.ds(t0, L)] = i2
        return carry

    lax.fori_loop(0, GROUPS, group, 0)
    for e in range(NE):
        pltpu.sync_copy(cw_v.at[e], cw_hbm.at[e, pl.ds(base, TPW)])
    pltpu.sync_copy(i1_v, idx_hbm.at[0, pl.ds(base, TPW)])
    pltpu.sync_copy(i2_v, idx_hbm.at[1, pl.ds(base, TPW)])


def kernel(x, W, b):
    logits_t = _logits_tc(x, W, b)
    cw_t, idx_t = _route_sc(logits_t)
    cw = cw_t.T[..., None]
    idx = idx_t.T
    return (cw, idx, jnp.float32(0.0))


# speedup vs baseline: 1.4881x; 1.0235x over previous
import functools

import jax
import jax.numpy as jnp
from jax import lax
from jax.experimental import pallas as pl
from jax.experimental.pallas import tpu as pltpu
from jax.experimental.pallas import tpu_sc as plsc

N_TOK = 16384
DM = 2048
NE = 16
TN = 2048

NW = 32
TPW = N_TOK // NW
GROUPS = TPW // 16
L = 16


def _logits_body(x_ref, w_ref, b_ref, lg_ref):
    lg = jax.lax.dot_general(
        x_ref[...], w_ref[...], (((1,), (1,)), ((), ())),
        preferred_element_type=jnp.float32,
    ) + b_ref[...]
    lg_ref[...] = lg.T


def _logits_tc(x, W, b):
    grid = N_TOK // TN
    return pl.pallas_call(
        _logits_body,
        grid=(grid,),
        in_specs=[
            pl.BlockSpec((TN, DM), lambda i: (i, 0)),
            pl.BlockSpec((NE, DM), lambda i: (0, 0)),
            pl.BlockSpec((1, NE), lambda i: (0, 0)),
        ],
        out_specs=pl.BlockSpec((NE, TN), lambda i: (0, i)),
        out_shape=jax.ShapeDtypeStruct((NE, N_TOK), jnp.float32),
        compiler_params=pltpu.CompilerParams(
            dimension_semantics=("arbitrary",)
        ),
    )(x, W, b.reshape(1, NE))


_mesh = plsc.VectorSubcoreMesh(core_axis_name="c", subcore_axis_name="s")


@functools.partial(
    pl.kernel,
    out_type=[
        jax.ShapeDtypeStruct((NE, N_TOK), jnp.float32),
        jax.ShapeDtypeStruct((2, N_TOK), jnp.int32),
    ],
    mesh=_mesh,
    scratch_types=[
        pltpu.VMEM((NE, TPW), jnp.float32),
        pltpu.VMEM((NE, TPW), jnp.float32),
        pltpu.VMEM((TPW,), jnp.int32),
        pltpu.VMEM((TPW,), jnp.int32),
    ],
    compiler_params=pltpu.CompilerParams(needs_layout_passes=False),
)
def _route_sc(lg_hbm, cw_hbm, idx_hbm, lg_v, cw_v, i1_v, i2_v):
    wid = lax.axis_index("s") * 2 + lax.axis_index("c")
    base = wid * TPW
    pltpu.sync_copy(lg_hbm.at[:, pl.ds(base, TPW)], lg_v)

    neg = jnp.full((L,), -3.4e38, jnp.float32)
    zero_i = jnp.zeros((L,), jnp.int32)
    zero_f = jnp.zeros((L,), jnp.float32)

    def group(g, carry):
        t0 = g * L
        m1, m2 = neg, neg
        i1, i2 = zero_i, zero_i
        for e in range(NE):
            esp = jnp.full((L,), e, jnp.int32)
            le = lg_v[e, pl.ds(t0, L)]
            gt1 = le > m1
            gt2 = le > m2
            i2 = jnp.where(gt1, i1, jnp.where(gt2, esp, i2))
            m2 = jnp.where(gt1, m1, jnp.where(gt2, le, m2))
            i1 = jnp.where(gt1, esp, i1)
            m1 = jnp.where(gt1, le, m1)
        w1 = 1.0 / (1.0 + jnp.exp(m2 - m1))
        w2 = 1.0 - w1
        for e in range(NE):
            esp = jnp.full((L,), e, jnp.int32)
            val = jnp.where(i1 == esp, w1, jnp.where(i2 == esp, w2, zero_f))
            cw_v[e, pl.ds(t0, L)] = val
        i1_v[pl.ds(t0, L)] = i1
        i2_v[pl.ds(t0, L)] = i2
        return carry

    lax.fori_loop(0, GROUPS, group, 0)
    pltpu.sync_copy(cw_v, cw_hbm.at[:, pl.ds(base, TPW)])
    pltpu.sync_copy(i1_v, idx_hbm.at[0, pl.ds(base, TPW)])
    pltpu.sync_copy(i2_v, idx_hbm.at[1, pl.ds(base, TPW)])


def kernel(x, W, b):
    logits_t = _logits_tc(x, W, b)
    cw_t, idx_t = _route_sc(logits_t)
    cw = cw_t.T[..., None]
    idx = idx_t.T
    return (cw, idx, jnp.float32(0.0))
